# baseline (speedup 1.0000x reference)
import jax
import jax.numpy as jnp
from jax.experimental import pallas as pl

_H, _W = 320, 800
_THR = 0.1
_HB = 32


def _decode_body(heat_ref, off_ref, err_ref, out_ref):
    h = heat_ref[...]
    ninf = jnp.full((_HB, 1), -jnp.inf, dtype=jnp.float32)
    lft = jnp.concatenate([ninf, h[:, :-1]], axis=1)
    rgt = jnp.concatenate([h[:, 1:], ninf], axis=1)
    hmax = jnp.maximum(jnp.maximum(lft, rgt), h)
    nms = jnp.where(hmax == h, h, 0.0)
    m = nms > _THR

    xs = jax.lax.broadcasted_iota(jnp.int32, (_HB, _W), 1).astype(jnp.float32)
    ys = (pl.program_id(0) * _HB).astype(jnp.float32) + jax.lax.broadcasted_iota(
        jnp.int32, (_HB, _W), 0
    ).astype(jnp.float32)

    out_ref[0] = nms
    out_ref[1] = jnp.where(m, xs + off_ref[0], 0.0)
    out_ref[2] = jnp.where(m, ys + off_ref[1], 0.0)
    out_ref[3] = jnp.where(m, xs + err_ref[0], 0.0)
    out_ref[4] = jnp.where(m, ys + err_ref[1], 0.0)


@jax.jit
def _decode(heat2d, off, err):
    return pl.pallas_call(
        _decode_body,
        grid=(_H // _HB,),
        in_specs=[
            pl.BlockSpec((_HB, _W), lambda i: (i, 0)),
            pl.BlockSpec((2, _HB, _W), lambda i: (0, i, 0)),
            pl.BlockSpec((2, _HB, _W), lambda i: (0, i, 0)),
        ],
        out_specs=pl.BlockSpec((5, _HB, _W), lambda i: (0, i, 0)),
        out_shape=jax.ShapeDtypeStruct((5, _H, _W), jnp.float32),
    )(heat2d, off, err)


def kernel(heat, offset, error):
    hf = heat.reshape(_H, _W)
    off = offset.reshape(2, _H, _W)
    err = error.reshape(2, _H, _W)
    out5 = _decode(hf, off, err)
    return jnp.transpose(out5, (1, 2, 0))


# speedup vs baseline: 1.8173x; 1.8173x over previous
import jax
import jax.numpy as jnp
from jax.experimental import pallas as pl

_H, _W = 320, 800
_THR = 0.1
_HB = 160


def _decode_body(heat_ref, off_ref, err_ref, out_ref):
    h = heat_ref[...]
    ninf = jnp.full((_HB, 1), -jnp.inf, dtype=jnp.float32)
    lft = jnp.concatenate([ninf, h[:, :-1]], axis=1)
    rgt = jnp.concatenate([h[:, 1:], ninf], axis=1)
    hmax = jnp.maximum(jnp.maximum(lft, rgt), h)
    nms = jnp.where(hmax == h, h, 0.0)
    m = nms > _THR

    xs = jax.lax.broadcasted_iota(jnp.int32, (_HB, _W), 1).astype(jnp.float32)
    ys = (pl.program_id(0) * _HB).astype(jnp.float32) + jax.lax.broadcasted_iota(
        jnp.int32, (_HB, _W), 0
    ).astype(jnp.float32)

    out_ref[0] = nms
    out_ref[1] = jnp.where(m, xs + off_ref[0], 0.0)
    out_ref[2] = jnp.where(m, ys + off_ref[1], 0.0)
    out_ref[3] = jnp.where(m, xs + err_ref[0], 0.0)
    out_ref[4] = jnp.where(m, ys + err_ref[1], 0.0)


@jax.jit
def _decode(heat2d, off, err):
    return pl.pallas_call(
        _decode_body,
        grid=(_H // _HB,),
        in_specs=[
            pl.BlockSpec((_HB, _W), lambda i: (i, 0)),
            pl.BlockSpec((2, _HB, _W), lambda i: (0, i, 0)),
            pl.BlockSpec((2, _HB, _W), lambda i: (0, i, 0)),
        ],
        out_specs=pl.BlockSpec((5, _HB, _W), lambda i: (0, i, 0)),
        out_shape=jax.ShapeDtypeStruct((5, _H, _W), jnp.float32),
    )(heat2d, off, err)


def kernel(heat, offset, error):
    hf = heat.reshape(_H, _W)
    off = offset.reshape(2, _H, _W)
    err = error.reshape(2, _H, _W)
    out5 = _decode(hf, off, err)
    return jnp.transpose(out5, (1, 2, 0))
